# baseline (speedup 1.0000x reference)
import jax
import jax.numpy as jnp
from jax import lax
from jax.experimental import pallas as pl
from jax.experimental.pallas import tpu as pltpu
from jax.experimental.pallas import tpu_sc as plsc

_HIDDEN = 768
_INTER = 1536
_E = 64
_N_TOK = 2048
_TB = 256
_TM = 64
_NPAD = _N_TOK + 7 * _E + _TM


def _router_body(x_ref, wr_ref, idx_ref, aux_ref, tile_ref, cnt_ref, prb_ref):
    i = pl.program_id(0)
    xb = x_ref[...]
    wr = wr_ref[...]
    logits = jax.lax.dot_general(xb, wr, (((1,), (1,)), ((), ())),
                                 preferred_element_type=jnp.float32)
    m = jnp.max(logits, axis=1, keepdims=True)
    iota = jax.lax.broadcasted_iota(jnp.int32, (_TB, _E), 1)
    eidx = jnp.min(jnp.where(logits == m, iota, _E), axis=1).astype(jnp.int32)
    p = jnp.exp(logits - m)
    p = p / jnp.sum(p, axis=1, keepdims=True)
    onehot = (iota == eidx[:, None]).astype(jnp.float32)

    idx_ref[pl.ds(i * _TB, _TB)] = eidx

    half = _TB // 2
    c0 = jnp.sum(onehot[:half], axis=0)
    c1 = jnp.sum(onehot[half:], axis=0)
    tile_ref[...] = jnp.stack([c0, c1]).astype(jnp.int32).reshape(1, 2, _E)

    @pl.when(i == 0)
    def _():
        cnt_ref[...] = jnp.zeros_like(cnt_ref)
        prb_ref[...] = jnp.zeros_like(prb_ref)

    cnt_ref[...] = cnt_ref[...] + jnp.sum(onehot, axis=0).reshape(1, _E)
    prb_ref[...] = prb_ref[...] + jnp.sum(p, axis=0).reshape(1, _E)

    @pl.when(i == pl.num_programs(0) - 1)
    def _():
        aux = jnp.sum(cnt_ref[...] * prb_ref[...])
        aux_ref[0, 0] = aux * (_E / (_N_TOK * _N_TOK))


def _router(x, wr):
    eidx, aux, tiles = pl.pallas_call(
        _router_body,
        grid=(_N_TOK // _TB,),
        in_specs=[
            pl.BlockSpec((_TB, _HIDDEN), lambda i: (i, 0)),
            pl.BlockSpec((_E, _HIDDEN), lambda i: (0, 0)),
        ],
        out_specs=[
            pl.BlockSpec((_N_TOK,), lambda i: (0,)),
            pl.BlockSpec(memory_space=pltpu.SMEM),
            pl.BlockSpec((1, 2, _E), lambda i: (i, 0, 0)),
        ],
        out_shape=[
            jax.ShapeDtypeStruct((_N_TOK,), jnp.int32),
            jax.ShapeDtypeStruct((1, 1), jnp.float32),
            jax.ShapeDtypeStruct((_N_TOK // _TB, 2, _E), jnp.int32),
        ],
        scratch_shapes=[
            pltpu.VMEM((1, _E), jnp.float32),
            pltpu.VMEM((1, _E), jnp.float32),
        ],
    )(x, wr)
    return eidx, aux, tiles.reshape(_N_TOK // 128, _E)


def _ffn_body(off_ref, xs_ref, w1_ref, w2_ref, out_ref):
    e = pl.program_id(0)
    start = pl.multiple_of(off_ref[e], 8)
    end = off_ref[e + 1]
    nch = (end - start + _TM - 1) // _TM
    w1 = w1_ref[0]
    w2 = w2_ref[0]

    def body(k, carry):
        base = pl.multiple_of(start + k * _TM, 8)
        rows = xs_ref[pl.ds(base, _TM), :].astype(jnp.bfloat16)
        h = jax.lax.dot_general(rows, w1, (((1,), (1,)), ((), ())),
                                preferred_element_type=jnp.float32)
        h = (h * jax.nn.sigmoid(h)).astype(jnp.bfloat16)
        o = jax.lax.dot_general(h, w2, (((1,), (1,)), ((), ())),
                                preferred_element_type=jnp.float32)
        out_ref[pl.ds(base, _TM), :] = o
        return carry

    jax.lax.fori_loop(0, nch, body, 0)


def _grouped_ffn(off, xs, w1, w2):
    grid_spec = pltpu.PrefetchScalarGridSpec(
        num_scalar_prefetch=1,
        grid=(_E,),
        in_specs=[
            pl.BlockSpec((_NPAD, _HIDDEN), lambda e, off: (0, 0)),
            pl.BlockSpec((1, _INTER, _HIDDEN), lambda e, off: (e, 0, 0)),
            pl.BlockSpec((1, _HIDDEN, _INTER), lambda e, off: (e, 0, 0)),
        ],
        out_specs=pl.BlockSpec((_NPAD, _HIDDEN), lambda e, off: (0, 0)),
    )
    return pl.pallas_call(
        _ffn_body,
        grid_spec=grid_spec,
        out_shape=jax.ShapeDtypeStruct((_NPAD, _HIDDEN), jnp.float32),
    )(off, xs, w1, w2)


_NC = 2
_NS = 16
_L = 16
_TPT = _N_TOK // _NS
_HALF = _TPT // _NC


def _bin_body(idx_hbm, cnt_hbm, x_hbm, xs_hbm, off_hbm, pos_hbm,
              idx_v, rank_v, run_v, allc_v, base_v, posh_v,
              poff_v, xrow_v, sem):
    c = lax.axis_index("c")
    s = lax.axis_index("s")
    zero16 = jnp.zeros((_L,), jnp.int32)
    iota16 = lax.iota(jnp.int32, _L)

    pltpu.sync_copy(idx_hbm.at[pl.ds(s * _TPT, _TPT)], idx_v)
    pltpu.sync_copy(cnt_hbm, allc_v)
    for g in range(_E // _L):
        run_v[pl.ds(g * _L, _L)] = zero16

    for tv in range(_TPT // _L):
        idx16 = idx_v[pl.ds(tv * _L, _L)]
        rank16 = zero16
        cnt16 = zero16
        for j in range(_L):
            xj = jnp.take(idx16, jnp.full((_L,), j, jnp.int32))
            one = (idx16 == xj).astype(jnp.int32)
            rank16 = rank16 + jnp.where(iota16 > j, one, zero16)
            cnt16 = cnt16 + one
        base16 = plsc.load_gather(run_v, [idx16])
        rank_v[pl.ds(tv * _L, _L)] = base16 + rank16
        plsc.store_scatter(run_v, [idx16], base16 + cnt16,
                           mask=rank16 == cnt16 - 1)

    carry = jnp.int32(0)
    for g in range(_E // _L):
        tot = zero16
        pre = zero16
        for w in range(_NS):
            row = allc_v[w, pl.ds(g * _L, _L)]
            tot = tot + row
            wvec = jnp.full((_L,), w, jnp.int32)
            pre = pre + jnp.where(wvec < s, row, zero16)
        cpad = jnp.bitwise_and(tot + 7, -8)
        incl = plsc.cumsum(cpad)
        excl = incl - cpad + carry
        poff_v[pl.ds(g * _L, _L)] = excl
        base_v[pl.ds(g * _L, _L)] = excl + pre
        carry = carry + jnp.sum(cpad, axis=0)
    poff_v[pl.ds(_E, _L)] = jnp.where(iota16 == 0, carry, 0)

    for tv in range(_TPT // _L):
        idx16 = idx_v[pl.ds(tv * _L, _L)]
        b16 = plsc.load_gather(base_v, [idx16])
        rank_v[pl.ds(tv * _L, _L)] = b16 + rank_v[pl.ds(tv * _L, _L)]
    for hv in range(_HALF // _L):
        posh_v[pl.ds(hv * _L, _L)] = rank_v[pl.ds(c * _HALF + hv * _L, _L)]
    pltpu.sync_copy(posh_v, pos_hbm.at[pl.ds(s * _TPT + c * _HALF, _HALF)])

    pltpu.sync_copy(x_hbm.at[pl.ds(s * _TPT + c * _HALF, _HALF)], xrow_v)
    pltpu.async_copy(xrow_v, xs_hbm.at[posh_v], sem).wait()

    @pl.when(jnp.logical_and(c == 0, s == 0))
    def _():
        pltpu.sync_copy(poff_v, off_hbm)


def _sc_bin(eidx, cnt_tile, x):
    mesh = plsc.VectorSubcoreMesh(core_axis_name="c", subcore_axis_name="s",
                                  num_cores=_NC, num_subcores=_NS)
    f = pl.kernel(
        _bin_body,
        out_type=[
            jax.ShapeDtypeStruct((_NPAD, _HIDDEN), jnp.float32),
            jax.ShapeDtypeStruct((80,), jnp.int32),
            jax.ShapeDtypeStruct((_N_TOK,), jnp.int32),
        ],
        mesh=mesh,
        compiler_params=pltpu.CompilerParams(needs_layout_passes=False),
        scratch_types=[
            pltpu.VMEM((_TPT,), jnp.int32),
            pltpu.VMEM((_TPT,), jnp.int32),
            pltpu.VMEM((_E,), jnp.int32),
            pltpu.VMEM((_NS, _E), jnp.int32),
            pltpu.VMEM((_E,), jnp.int32),
            pltpu.VMEM((_HALF,), jnp.int32),
            pltpu.VMEM((80,), jnp.int32),
            pltpu.VMEM((_HALF, _HIDDEN), jnp.float32),
            pltpu.SemaphoreType.DMA,
        ],
    )
    return f(eidx, cnt_tile, x)


def _unsort_body(o_hbm, pos_hbm, out_hbm, posd_v, rows_v, sem):
    c = lax.axis_index("c")
    s = lax.axis_index("s")
    wid = s * _NC + c
    pltpu.sync_copy(pos_hbm.at[pl.ds(wid * _HALF, _HALF)], posd_v)
    pltpu.async_copy(o_hbm.at[posd_v], rows_v, sem).wait()
    pltpu.sync_copy(rows_v, out_hbm.at[pl.ds(wid * _HALF, _HALF)])


def _sc_unsort(o_sorted, pos):
    mesh = plsc.VectorSubcoreMesh(core_axis_name="c", subcore_axis_name="s",
                                  num_cores=_NC, num_subcores=_NS)
    f = pl.kernel(
        _unsort_body,
        out_type=jax.ShapeDtypeStruct((_N_TOK, _HIDDEN), jnp.float32),
        mesh=mesh,
        scratch_types=[
            pltpu.VMEM((_HALF,), jnp.int32),
            pltpu.VMEM((_HALF, _HIDDEN), jnp.float32),
            pltpu.SemaphoreType.DMA,
        ],
    )
    return f(o_sorted, pos)


def kernel(x, Wr, W1, W2):
    eidx, aux, tiles = _router(x, Wr)
    xs, off, pos = _sc_bin(eidx, tiles, x)
    o_sorted = _grouped_ffn(off, xs, W1.astype(jnp.bfloat16),
                            W2.astype(jnp.bfloat16))
    out = _sc_unsort(o_sorted, pos)
    return (out, aux[0, 0])


# speedup vs baseline: 1.9223x; 1.9223x over previous
import jax
import jax.numpy as jnp
from jax import lax
from jax.experimental import pallas as pl
from jax.experimental.pallas import tpu as pltpu
from jax.experimental.pallas import tpu_sc as plsc

_HIDDEN = 768
_INTER = 1536
_E = 64
_N_TOK = 2048
_TB = 256
_TM = 64
_NPAD = _N_TOK + 7 * _E + _TM


def _router_body(x_ref, wr_ref, idx_ref, aux_ref, tile_ref, cnt_ref, prb_ref):
    i = pl.program_id(0)
    xb = x_ref[...]
    wr = wr_ref[...]
    logits = jax.lax.dot_general(xb, wr, (((1,), (1,)), ((), ())),
                                 preferred_element_type=jnp.float32)
    m = jnp.max(logits, axis=1, keepdims=True)
    iota = jax.lax.broadcasted_iota(jnp.int32, (_TB, _E), 1)
    eidx = jnp.min(jnp.where(logits == m, iota, _E), axis=1).astype(jnp.int32)
    p = jnp.exp(logits - m)
    p = p / jnp.sum(p, axis=1, keepdims=True)
    onehot = (iota == eidx[:, None]).astype(jnp.float32)

    idx_ref[pl.ds(i * _TB, _TB)] = eidx

    half = _TB // 2
    c0 = jnp.sum(onehot[:half], axis=0)
    c1 = jnp.sum(onehot[half:], axis=0)
    tile_ref[...] = jnp.stack([c0, c1]).astype(jnp.int32).reshape(1, 2, _E)

    @pl.when(i == 0)
    def _():
        cnt_ref[...] = jnp.zeros_like(cnt_ref)
        prb_ref[...] = jnp.zeros_like(prb_ref)

    cnt_ref[...] = cnt_ref[...] + jnp.sum(onehot, axis=0).reshape(1, _E)
    prb_ref[...] = prb_ref[...] + jnp.sum(p, axis=0).reshape(1, _E)

    @pl.when(i == pl.num_programs(0) - 1)
    def _():
        aux = jnp.sum(cnt_ref[...] * prb_ref[...])
        aux_ref[0, 0] = aux * (_E / (_N_TOK * _N_TOK))


def _router(x, wr):
    eidx, aux, tiles = pl.pallas_call(
        _router_body,
        grid=(_N_TOK // _TB,),
        in_specs=[
            pl.BlockSpec((_TB, _HIDDEN), lambda i: (i, 0)),
            pl.BlockSpec((_E, _HIDDEN), lambda i: (0, 0)),
        ],
        out_specs=[
            pl.BlockSpec((_N_TOK,), lambda i: (0,)),
            pl.BlockSpec(memory_space=pltpu.SMEM),
            pl.BlockSpec((1, 2, _E), lambda i: (i, 0, 0)),
        ],
        out_shape=[
            jax.ShapeDtypeStruct((_N_TOK,), jnp.int32),
            jax.ShapeDtypeStruct((1, 1), jnp.float32),
            jax.ShapeDtypeStruct((_N_TOK // _TB, 2, _E), jnp.int32),
        ],
        scratch_shapes=[
            pltpu.VMEM((1, _E), jnp.float32),
            pltpu.VMEM((1, _E), jnp.float32),
        ],
    )(x, wr)
    return eidx, aux, tiles.reshape(_N_TOK // 128, _E)


_EPB = 2


def _ffn_body(off_ref, xs_ref, w1_ref, w2_ref, out_ref):
    g = pl.program_id(0)
    for sub in range(_EPB):
        e = g * _EPB + sub
        start = pl.multiple_of(off_ref[e], 8)
        end = off_ref[e + 1]
        nch = (end - start + _TM - 1) // _TM
        w1 = w1_ref[sub]
        w2 = w2_ref[sub]

        def body(k, carry, w1=w1, w2=w2, start=start):
            base = pl.multiple_of(start + k * _TM, 8)
            rows = xs_ref[pl.ds(base, _TM), :]
            h = jax.lax.dot_general(rows, w1, (((1,), (1,)), ((), ())),
                                    preferred_element_type=jnp.float32)
            h = h * jax.nn.sigmoid(h)
            o = jax.lax.dot_general(h, w2, (((1,), (1,)), ((), ())),
                                    preferred_element_type=jnp.float32)
            out_ref[pl.ds(base, _TM), :] = o
            return carry

        jax.lax.fori_loop(0, nch, body, 0)


def _grouped_ffn(off, xs, w1, w2):
    grid_spec = pltpu.PrefetchScalarGridSpec(
        num_scalar_prefetch=1,
        grid=(_E // _EPB,),
        in_specs=[
            pl.BlockSpec((_NPAD, _HIDDEN), lambda g, off: (0, 0)),
            pl.BlockSpec((_EPB, _INTER, _HIDDEN), lambda g, off: (g, 0, 0)),
            pl.BlockSpec((_EPB, _HIDDEN, _INTER), lambda g, off: (g, 0, 0)),
        ],
        out_specs=pl.BlockSpec((_NPAD, _HIDDEN), lambda g, off: (0, 0)),
    )
    return pl.pallas_call(
        _ffn_body,
        grid_spec=grid_spec,
        out_shape=jax.ShapeDtypeStruct((_NPAD, _HIDDEN), jnp.float32),
        compiler_params=pltpu.CompilerParams(
            vmem_limit_bytes=100 * 1024 * 1024),
    )(off, xs, w1, w2)


_NC = 2
_NS = 16
_L = 16
_TPT = _N_TOK // _NS
_HALF = _TPT // _NC


def _bin_body(idx_hbm, cnt_hbm, x_hbm, xs_hbm, off_hbm, pos_hbm,
              idx_v, rank_v, run_v, allc_v, base_v, posh_v,
              poff_v, xrow_v, sem):
    c = lax.axis_index("c")
    s = lax.axis_index("s")
    zero16 = jnp.zeros((_L,), jnp.int32)
    iota16 = lax.iota(jnp.int32, _L)

    pltpu.sync_copy(idx_hbm.at[pl.ds(s * _TPT, _TPT)], idx_v)
    pltpu.sync_copy(cnt_hbm, allc_v)
    for g in range(_E // _L):
        run_v[pl.ds(g * _L, _L)] = zero16

    for tv in range(_TPT // _L):
        idx16 = idx_v[pl.ds(tv * _L, _L)]
        rank16 = zero16
        cnt16 = zero16
        for j in range(_L):
            xj = jnp.take(idx16, jnp.full((_L,), j, jnp.int32))
            one = (idx16 == xj).astype(jnp.int32)
            rank16 = rank16 + jnp.where(iota16 > j, one, zero16)
            cnt16 = cnt16 + one
        base16 = plsc.load_gather(run_v, [idx16])
        rank_v[pl.ds(tv * _L, _L)] = base16 + rank16
        plsc.store_scatter(run_v, [idx16], base16 + cnt16,
                           mask=rank16 == cnt16 - 1)

    carry = jnp.int32(0)
    for g in range(_E // _L):
        tot = zero16
        pre = zero16
        for w in range(_NS):
            row = allc_v[w, pl.ds(g * _L, _L)]
            tot = tot + row
            wvec = jnp.full((_L,), w, jnp.int32)
            pre = pre + jnp.where(wvec < s, row, zero16)
        cpad = jnp.bitwise_and(tot + 7, -8)
        incl = plsc.cumsum(cpad)
        excl = incl - cpad + carry
        poff_v[pl.ds(g * _L, _L)] = excl
        base_v[pl.ds(g * _L, _L)] = excl + pre
        carry = carry + jnp.sum(cpad, axis=0)
    poff_v[pl.ds(_E, _L)] = jnp.where(iota16 == 0, carry, 0)

    for tv in range(_TPT // _L):
        idx16 = idx_v[pl.ds(tv * _L, _L)]
        b16 = plsc.load_gather(base_v, [idx16])
        rank_v[pl.ds(tv * _L, _L)] = b16 + rank_v[pl.ds(tv * _L, _L)]
    for hv in range(_HALF // _L):
        posh_v[pl.ds(hv * _L, _L)] = rank_v[pl.ds(c * _HALF + hv * _L, _L)]
    pltpu.sync_copy(posh_v, pos_hbm.at[pl.ds(s * _TPT + c * _HALF, _HALF)])

    pltpu.sync_copy(x_hbm.at[pl.ds(s * _TPT + c * _HALF, _HALF)], xrow_v)
    pltpu.async_copy(xrow_v, xs_hbm.at[posh_v], sem).wait()

    @pl.when(jnp.logical_and(c == 0, s == 0))
    def _():
        pltpu.sync_copy(poff_v, off_hbm)


def _sc_bin(eidx, cnt_tile, x):
    mesh = plsc.VectorSubcoreMesh(core_axis_name="c", subcore_axis_name="s",
                                  num_cores=_NC, num_subcores=_NS)
    f = pl.kernel(
        _bin_body,
        out_type=[
            jax.ShapeDtypeStruct((_NPAD, _HIDDEN), jnp.float32),
            jax.ShapeDtypeStruct((80,), jnp.int32),
            jax.ShapeDtypeStruct((_N_TOK,), jnp.int32),
        ],
        mesh=mesh,
        compiler_params=pltpu.CompilerParams(needs_layout_passes=False),
        scratch_types=[
            pltpu.VMEM((_TPT,), jnp.int32),
            pltpu.VMEM((_TPT,), jnp.int32),
            pltpu.VMEM((_E,), jnp.int32),
            pltpu.VMEM((_NS, _E), jnp.int32),
            pltpu.VMEM((_E,), jnp.int32),
            pltpu.VMEM((_HALF,), jnp.int32),
            pltpu.VMEM((80,), jnp.int32),
            pltpu.VMEM((_HALF, _HIDDEN), jnp.float32),
            pltpu.SemaphoreType.DMA,
        ],
    )
    return f(eidx, cnt_tile, x)


def _unsort_body(o_hbm, pos_hbm, out_hbm, posd_v, rows_v, sem):
    c = lax.axis_index("c")
    s = lax.axis_index("s")
    wid = s * _NC + c
    pltpu.sync_copy(pos_hbm.at[pl.ds(wid * _HALF, _HALF)], posd_v)
    pltpu.async_copy(o_hbm.at[posd_v], rows_v, sem).wait()
    pltpu.sync_copy(rows_v, out_hbm.at[pl.ds(wid * _HALF, _HALF)])


def _sc_unsort(o_sorted, pos):
    mesh = plsc.VectorSubcoreMesh(core_axis_name="c", subcore_axis_name="s",
                                  num_cores=_NC, num_subcores=_NS)
    f = pl.kernel(
        _unsort_body,
        out_type=jax.ShapeDtypeStruct((_N_TOK, _HIDDEN), jnp.float32),
        mesh=mesh,
        scratch_types=[
            pltpu.VMEM((_HALF,), jnp.int32),
            pltpu.VMEM((_HALF, _HIDDEN), jnp.float32),
            pltpu.SemaphoreType.DMA,
        ],
    )
    return f(o_sorted, pos)


def kernel(x, Wr, W1, W2):
    eidx, aux, tiles = _router(x, Wr)
    xs, off, pos = _sc_bin(eidx, tiles, x)
    o_sorted = _grouped_ffn(off, xs, W1, W2)
    out = _sc_unsort(o_sorted, pos)
    return (out, aux[0, 0])
